# baseline (device time: 34353 ns/iter reference)
import jax
import jax.numpy as jnp
from jax import lax
from jax.experimental import pallas as pl
from jax.experimental.pallas import tpu as pltpu

N_DEV = 16
N_TOK = 2048
D_IN = 512
D_OUT = 1024
E_LOCAL = 4
CHUNK = N_TOK // N_DEV
CAP = 32
PAY = D_OUT + CHUNK
GRP = 4

_HBM = pltpu.MemorySpace.HBM


def _tdot(a, b):
    return lax.dot_general(
        a, b, dimension_numbers=(((0,), (0,)), ((), ())),
        preferred_element_type=jnp.float32,
    )


def kernel(x, router_W, route_idx, expert_W, shared_W):
    rw_pad = jnp.pad(router_W, ((0, 0), (0, 64)))
    onehot = (route_idx == jnp.arange(64, dtype=jnp.int32)[None, :]).astype(
        jnp.bfloat16
    )

    def body(
        x_hbm,
        rw_ref,
        oh_ref,
        ew_hbm,
        sw_hbm,
        out_hbm,
        xv_ref,
        xb_ref,
        ewf_ref,
        ewb_ref,
        swf_ref,
        swb_ref,
        mp_ref,
        ov_ref,
        sb_ref,
        rb_ref,
        ssem,
        rsem,
        csem,
    ):
        d = lax.axis_index("i")

        cp_x = pltpu.make_async_copy(x_hbm, xv_ref, csem.at[0])
        cp_ew = pltpu.make_async_copy(ew_hbm, ewf_ref, csem.at[1])
        cp_sw = pltpu.make_async_copy(sw_hbm, swf_ref, csem.at[2])
        cp_x.start()
        cp_ew.start()
        cp_sw.start()

        bsem = pltpu.get_barrier_semaphore()
        for off in range(1, N_DEV):
            pl.semaphore_signal(
                bsem, inc=1,
                device_id=((d + off) % N_DEV,),
                device_id_type=pl.DeviceIdType.MESH,
            )

        cp_x.wait()
        xb_ref[...] = xv_ref[...].astype(jnp.bfloat16)
        scores = jnp.dot(xv_ref[...], rw_ref[...],
                         preferred_element_type=jnp.float32)[:, :64]
        smax = jnp.max(scores, axis=-1, keepdims=True)
        p = jnp.exp(scores - smax)
        probs = p / jnp.sum(p, axis=-1, keepdims=True)
        mp_ref[...] = probs * oh_ref[...].astype(jnp.float32)

        r0 = lax.broadcasted_iota(jnp.int32, (CHUNK, CHUNK), 0)
        r1 = lax.broadcasted_iota(jnp.int32, (CHUNK, CHUNK), 1)
        ltri = (r1 < r0).astype(jnp.bfloat16)
        ident = (r1 == r0).astype(jnp.bfloat16)

        def compact(c):
            rows = pl.ds(c * CHUNK, CHUNK)
            xc = xb_ref[rows, :]
            mpc = mp_ref[rows, :]
            ci = lax.broadcasted_iota(jnp.int32, (CHUNK, 64), 1)
            cos = []
            for e in range(E_LOCAL):
                ge = d * E_LOCAL + e
                cos.append(jnp.sum(jnp.where(ci == ge, mpc, 0.0),
                                   axis=-1, keepdims=True))
            m = (cos[0] + cos[1] + cos[2] + cos[3]) > 0.0
            cume = jnp.dot(ltri, m.astype(jnp.bfloat16),
                           preferred_element_type=jnp.float32)
            si = lax.broadcasted_iota(jnp.int32, (CHUNK, CAP), 1)
            hit = (cume.astype(jnp.int32) == si) & m
            s_all = jnp.concatenate(
                [jnp.where(hit, co, 0.0).astype(jnp.bfloat16) for co in cos],
                axis=1,
            )
            xg_all = _tdot(s_all, xc).astype(jnp.bfloat16)
            xg_cat = jnp.concatenate(
                [xg_all[e * CAP:(e + 1) * CAP, :] for e in range(E_LOCAL)],
                axis=1,
            )
            s_t = _tdot(hit.astype(jnp.bfloat16),
                        ident).astype(jnp.bfloat16)
            return xg_cat, s_t

        pl.semaphore_wait(bsem, N_DEV - 1)

        rdmas = []
        for g in range(GRP):
            offs = [4 * g + 1 + j for j in range(4)]
            chunks = [(d + o) % N_DEV if o < N_DEV else d for o in offs]
            parts = [compact(c) for c in chunks]
            xg_stack = jnp.concatenate([pp[0] for pp in parts], axis=0)
            st_stack = jnp.concatenate([pp[1] for pp in parts], axis=0)
            if g == 0:
                cp_ew.wait()
                for e in range(E_LOCAL):
                    ewb_ref[pl.ds(e * D_IN, D_IN), :] = (
                        ewf_ref[e].astype(jnp.bfloat16)
                    )
            y_g = jnp.dot(xg_stack, ewb_ref[...],
                          preferred_element_type=jnp.float32)
            sb_ref[g] = jnp.concatenate(
                [y_g.astype(jnp.bfloat16), st_stack], axis=1)
            for j, off in enumerate(offs):
                if off >= N_DEV:
                    continue
                rdma = pltpu.make_async_remote_copy(
                    src_ref=sb_ref.at[g, pl.ds(j * CAP, CAP), :],
                    dst_ref=rb_ref.at[N_DEV - 1 - off],
                    send_sem=ssem.at[off - 1],
                    recv_sem=rsem.at[N_DEV - 1 - off],
                    device_id=((d + off) % N_DEV,),
                    device_id_type=pl.DeviceIdType.MESH,
                )
                rdma.start()
                rdmas.append(rdma)

        cp_sw.wait()
        swb_ref[...] = swf_ref[...].astype(jnp.bfloat16)
        tot = jnp.dot(xb_ref[pl.ds(d * CHUNK, CHUNK), :], swb_ref[...],
                      preferred_element_type=jnp.float32)

        def decomp(blk):
            return _tdot(blk[:, D_OUT:], blk[:, :D_OUT])

        for grp_j in range(3):
            for j in range(4 * grp_j, 4 * grp_j + 4):
                rdmas[j].wait()
            lo = 14 - (4 * grp_j + 3)
            blk = jnp.reshape(rb_ref[pl.ds(lo, 4), :, :], (4 * CAP, PAY))
            tot += decomp(blk)
        for j in range(12, 15):
            rdmas[j].wait()
        last = jnp.concatenate(
            [jnp.reshape(rb_ref[pl.ds(0, 3), :, :], (3 * CAP, PAY)),
             sb_ref[3, pl.ds(3 * CAP, CAP), :]],
            axis=0,
        )
        tot += decomp(last)

        ov_ref[...] = tot
        cp_out = pltpu.make_async_copy(ov_ref, out_hbm, csem.at[0])
        cp_out.start()
        cp_out.wait()

    return pl.pallas_call(
        body,
        out_shape=jax.ShapeDtypeStruct((CHUNK, D_OUT), jnp.float32),
        in_specs=[
            pl.BlockSpec(memory_space=_HBM),
            pl.BlockSpec(memory_space=pltpu.VMEM),
            pl.BlockSpec(memory_space=pltpu.VMEM),
            pl.BlockSpec(memory_space=_HBM),
            pl.BlockSpec(memory_space=_HBM),
        ],
        out_specs=pl.BlockSpec(memory_space=_HBM),
        scratch_shapes=[
            pltpu.VMEM((N_TOK, D_IN), jnp.float32),
            pltpu.VMEM((N_TOK, D_IN), jnp.bfloat16),
            pltpu.VMEM((E_LOCAL, D_IN, D_OUT), jnp.float32),
            pltpu.VMEM((E_LOCAL * D_IN, D_OUT), jnp.bfloat16),
            pltpu.VMEM((D_IN, D_OUT), jnp.float32),
            pltpu.VMEM((D_IN, D_OUT), jnp.bfloat16),
            pltpu.VMEM((N_TOK, 64), jnp.float32),
            pltpu.VMEM((CHUNK, D_OUT), jnp.float32),
            pltpu.VMEM((GRP, GRP * CAP, PAY), jnp.bfloat16),
            pltpu.VMEM((N_DEV - 1, CAP, PAY), jnp.bfloat16),
            pltpu.SemaphoreType.DMA((N_DEV - 1,)),
            pltpu.SemaphoreType.DMA((N_DEV - 1,)),
            pltpu.SemaphoreType.DMA((3,)),
        ],
        compiler_params=pltpu.CompilerParams(collective_id=0),
    )(x, rw_pad, onehot, expert_W, shared_W)


# device time: 26456 ns/iter; 1.2985x vs baseline; 1.2985x over previous
import jax
import jax.numpy as jnp
from jax import lax
from jax.experimental import pallas as pl
from jax.experimental.pallas import tpu as pltpu

N_DEV = 16
N_TOK = 2048
D_IN = 512
D_OUT = 1024
E_LOCAL = 4
CHUNK = N_TOK // N_DEV
CAP = 32
PAY = D_OUT + CHUNK
GRP = 4

_HBM = pltpu.MemorySpace.HBM


def _tdot(a, b):
    return lax.dot_general(
        a, b, dimension_numbers=(((0,), (0,)), ((), ())),
        preferred_element_type=jnp.float32,
    )


def kernel(x, router_W, route_idx, expert_W, shared_W):
    rw_pad = jnp.pad(router_W, ((0, 0), (0, 64)))
    onehot = (route_idx == jnp.arange(64, dtype=jnp.int32)[None, :]).astype(
        jnp.bfloat16
    )
    x = pltpu.with_memory_space_constraint(x, _HBM)
    expert_W = pltpu.with_memory_space_constraint(expert_W, _HBM)
    shared_W = pltpu.with_memory_space_constraint(shared_W, _HBM)

    def body(
        x_hbm,
        rw_ref,
        oh_ref,
        ew_hbm,
        sw_hbm,
        out_hbm,
        xv_ref,
        xb_ref,
        ewf_ref,
        ewb_ref,
        swf_ref,
        swb_ref,
        mp_ref,
        ov_ref,
        sb_ref,
        rb_ref,
        ssem,
        rsem,
        csem,
    ):
        d = lax.axis_index("i")

        cp_x = pltpu.make_async_copy(x_hbm, xv_ref, csem.at[0])
        cp_ew = pltpu.make_async_copy(ew_hbm, ewf_ref, csem.at[1])
        cp_sw = pltpu.make_async_copy(sw_hbm, swf_ref, csem.at[2])
        cp_x.start()
        cp_ew.start()
        cp_sw.start()

        bsem = pltpu.get_barrier_semaphore()
        for off in range(1, N_DEV):
            pl.semaphore_signal(
                bsem, inc=1,
                device_id=((d + off) % N_DEV,),
                device_id_type=pl.DeviceIdType.MESH,
            )

        cp_x.wait()
        xb_ref[...] = xv_ref[...].astype(jnp.bfloat16)
        scores = jnp.dot(xv_ref[...], rw_ref[...],
                         preferred_element_type=jnp.float32)[:, :64]
        smax = jnp.max(scores, axis=-1, keepdims=True)
        p = jnp.exp(scores - smax)
        probs = p / jnp.sum(p, axis=-1, keepdims=True)
        mp_ref[...] = probs * oh_ref[...].astype(jnp.float32)

        r0 = lax.broadcasted_iota(jnp.int32, (CHUNK, CHUNK), 0)
        r1 = lax.broadcasted_iota(jnp.int32, (CHUNK, CHUNK), 1)
        ltri = (r1 < r0).astype(jnp.bfloat16)
        ident = (r1 == r0).astype(jnp.bfloat16)

        def compact(c):
            rows = pl.ds(c * CHUNK, CHUNK)
            xc = xb_ref[rows, :]
            mpc = mp_ref[rows, :]
            ci = lax.broadcasted_iota(jnp.int32, (CHUNK, 64), 1)
            cos = []
            for e in range(E_LOCAL):
                ge = d * E_LOCAL + e
                cos.append(jnp.sum(jnp.where(ci == ge, mpc, 0.0),
                                   axis=-1, keepdims=True))
            m = (cos[0] + cos[1] + cos[2] + cos[3]) > 0.0
            cume = jnp.dot(ltri, m.astype(jnp.bfloat16),
                           preferred_element_type=jnp.float32)
            si = lax.broadcasted_iota(jnp.int32, (CHUNK, CAP), 1)
            hit = (cume.astype(jnp.int32) == si) & m
            s_all = jnp.concatenate(
                [jnp.where(hit, co, 0.0).astype(jnp.bfloat16) for co in cos],
                axis=1,
            )
            xg_all = _tdot(s_all, xc).astype(jnp.bfloat16)
            xg_cat = jnp.concatenate(
                [xg_all[e * CAP:(e + 1) * CAP, :] for e in range(E_LOCAL)],
                axis=1,
            )
            s_t = _tdot(hit.astype(jnp.bfloat16),
                        ident).astype(jnp.bfloat16)
            return xg_cat, s_t

        pl.semaphore_wait(bsem, N_DEV - 1)

        rdmas = []
        for g in range(GRP):
            offs = [4 * g + 1 + j for j in range(4)]
            chunks = [(d + o) % N_DEV if o < N_DEV else d for o in offs]
            parts = [compact(c) for c in chunks]
            xg_stack = jnp.concatenate([pp[0] for pp in parts], axis=0)
            st_stack = jnp.concatenate([pp[1] for pp in parts], axis=0)
            if g == 0:
                cp_ew.wait()
                for e in range(E_LOCAL):
                    ewb_ref[pl.ds(e * D_IN, D_IN), :] = (
                        ewf_ref[e].astype(jnp.bfloat16)
                    )
            y_g = jnp.dot(xg_stack, ewb_ref[...],
                          preferred_element_type=jnp.float32)
            sb_ref[g] = jnp.concatenate(
                [y_g.astype(jnp.bfloat16), st_stack], axis=1)
            for j, off in enumerate(offs):
                if off >= N_DEV:
                    continue
                rdma = pltpu.make_async_remote_copy(
                    src_ref=sb_ref.at[g, pl.ds(j * CAP, CAP), :],
                    dst_ref=rb_ref.at[N_DEV - 1 - off],
                    send_sem=ssem.at[off - 1],
                    recv_sem=rsem.at[N_DEV - 1 - off],
                    device_id=((d + off) % N_DEV,),
                    device_id_type=pl.DeviceIdType.MESH,
                )
                rdma.start()
                rdmas.append(rdma)

        cp_sw.wait()
        swb_ref[...] = swf_ref[...].astype(jnp.bfloat16)
        tot = jnp.dot(xb_ref[pl.ds(d * CHUNK, CHUNK), :], swb_ref[...],
                      preferred_element_type=jnp.float32)

        def decomp(blk):
            return _tdot(blk[:, D_OUT:], blk[:, :D_OUT])

        for grp_j in range(3):
            for j in range(4 * grp_j, 4 * grp_j + 4):
                rdmas[j].wait()
            lo = 14 - (4 * grp_j + 3)
            blk = jnp.reshape(rb_ref[pl.ds(lo, 4), :, :], (4 * CAP, PAY))
            tot += decomp(blk)
        for j in range(12, 15):
            rdmas[j].wait()
        last = jnp.concatenate(
            [jnp.reshape(rb_ref[pl.ds(0, 3), :, :], (3 * CAP, PAY)),
             sb_ref[3, pl.ds(3 * CAP, CAP), :]],
            axis=0,
        )
        tot += decomp(last)

        ov_ref[...] = tot
        cp_out = pltpu.make_async_copy(ov_ref, out_hbm, csem.at[0])
        cp_out.start()
        cp_out.wait()

    return pl.pallas_call(
        body,
        out_shape=jax.ShapeDtypeStruct((CHUNK, D_OUT), jnp.float32),
        in_specs=[
            pl.BlockSpec(memory_space=_HBM),
            pl.BlockSpec(memory_space=pltpu.VMEM),
            pl.BlockSpec(memory_space=pltpu.VMEM),
            pl.BlockSpec(memory_space=_HBM),
            pl.BlockSpec(memory_space=_HBM),
        ],
        out_specs=pl.BlockSpec(memory_space=_HBM),
        scratch_shapes=[
            pltpu.VMEM((N_TOK, D_IN), jnp.float32),
            pltpu.VMEM((N_TOK, D_IN), jnp.bfloat16),
            pltpu.VMEM((E_LOCAL, D_IN, D_OUT), jnp.float32),
            pltpu.VMEM((E_LOCAL * D_IN, D_OUT), jnp.bfloat16),
            pltpu.VMEM((D_IN, D_OUT), jnp.float32),
            pltpu.VMEM((D_IN, D_OUT), jnp.bfloat16),
            pltpu.VMEM((N_TOK, 64), jnp.float32),
            pltpu.VMEM((CHUNK, D_OUT), jnp.float32),
            pltpu.VMEM((GRP, GRP * CAP, PAY), jnp.bfloat16),
            pltpu.VMEM((N_DEV - 1, CAP, PAY), jnp.bfloat16),
            pltpu.SemaphoreType.DMA((N_DEV - 1,)),
            pltpu.SemaphoreType.DMA((N_DEV - 1,)),
            pltpu.SemaphoreType.DMA((3,)),
        ],
        compiler_params=pltpu.CompilerParams(collective_id=0),
    )(x, rw_pad, onehot, expert_W, shared_W)


# device time: 26176 ns/iter; 1.3124x vs baseline; 1.0107x over previous
import jax
import jax.numpy as jnp
from jax import lax
from jax.experimental import pallas as pl
from jax.experimental.pallas import tpu as pltpu

N_DEV = 16
N_TOK = 2048
D_IN = 512
D_OUT = 1024
E_LOCAL = 4
CHUNK = N_TOK // N_DEV
CAP = 32
PAY = D_OUT + CHUNK
GRP = 4

_HBM = pltpu.MemorySpace.HBM


def _tdot(a, b):
    return lax.dot_general(
        a, b, dimension_numbers=(((0,), (0,)), ((), ())),
        preferred_element_type=jnp.float32,
    )


def kernel(x, router_W, route_idx, expert_W, shared_W):
    rw_t = router_W.T
    onehot = (route_idx == jnp.arange(64, dtype=jnp.int32)[None, :]).astype(
        jnp.bfloat16
    )
    x = pltpu.with_memory_space_constraint(x, _HBM)
    expert_W = pltpu.with_memory_space_constraint(expert_W, _HBM)
    shared_W = pltpu.with_memory_space_constraint(shared_W, _HBM)

    def body(
        x_hbm,
        rw_ref,
        oh_ref,
        ew_hbm,
        sw_hbm,
        out_hbm,
        xv_ref,
        xb_ref,
        ewf_ref,
        ewb_ref,
        swf_ref,
        swb_ref,
        mp_ref,
        ov_ref,
        sb_ref,
        rb_ref,
        ssem,
        rsem,
        csem,
    ):
        d = lax.axis_index("i")

        cp_x = pltpu.make_async_copy(x_hbm, xv_ref, csem.at[0])
        cp_ew = pltpu.make_async_copy(ew_hbm, ewf_ref, csem.at[1])
        cp_sw = pltpu.make_async_copy(sw_hbm, swf_ref, csem.at[2])
        cp_x.start()
        cp_ew.start()
        cp_sw.start()

        bsem = pltpu.get_barrier_semaphore()
        for off in range(1, N_DEV):
            pl.semaphore_signal(
                bsem, inc=1,
                device_id=((d + off) % N_DEV,),
                device_id_type=pl.DeviceIdType.MESH,
            )

        cp_x.wait()
        xb_ref[...] = xv_ref[...].astype(jnp.bfloat16)
        scores = lax.dot_general(
            xv_ref[...], rw_ref[...],
            dimension_numbers=(((1,), (1,)), ((), ())),
            preferred_element_type=jnp.float32,
        )
        smax = jnp.max(scores, axis=-1, keepdims=True)
        p = jnp.exp(scores - smax)
        probs = p / jnp.sum(p, axis=-1, keepdims=True)
        mp_ref[...] = probs * oh_ref[...].astype(jnp.float32)

        r0 = lax.broadcasted_iota(jnp.int32, (CHUNK, CHUNK), 0)
        r1 = lax.broadcasted_iota(jnp.int32, (CHUNK, CHUNK), 1)
        ltri = (r1 < r0).astype(jnp.bfloat16)
        ident = (r1 == r0).astype(jnp.bfloat16)

        def compact(c):
            rows = pl.ds(c * CHUNK, CHUNK)
            xc = xb_ref[rows, :]
            mpc = mp_ref[rows, :]
            ci = lax.broadcasted_iota(jnp.int32, (CHUNK, 64), 1)
            cos = []
            for e in range(E_LOCAL):
                ge = d * E_LOCAL + e
                cos.append(jnp.sum(jnp.where(ci == ge, mpc, 0.0),
                                   axis=-1, keepdims=True))
            m = (cos[0] + cos[1] + cos[2] + cos[3]) > 0.0
            cume = jnp.dot(ltri, m.astype(jnp.bfloat16),
                           preferred_element_type=jnp.float32)
            si = lax.broadcasted_iota(jnp.int32, (CHUNK, CAP), 1)
            hit = (cume.astype(jnp.int32) == si) & m
            s_all = jnp.concatenate(
                [jnp.where(hit, co, 0.0).astype(jnp.bfloat16) for co in cos],
                axis=1,
            )
            xg_all = _tdot(s_all, xc).astype(jnp.bfloat16)
            xg_cat = jnp.concatenate(
                [xg_all[e * CAP:(e + 1) * CAP, :] for e in range(E_LOCAL)],
                axis=1,
            )
            s_t = _tdot(hit.astype(jnp.bfloat16),
                        ident).astype(jnp.bfloat16)
            return xg_cat, s_t

        pl.semaphore_wait(bsem, N_DEV - 1)

        rdmas = []
        for g in range(GRP):
            offs = [4 * g + 1 + j for j in range(4)]
            chunks = [(d + o) % N_DEV if o < N_DEV else d for o in offs]
            parts = [compact(c) for c in chunks]
            xg_stack = jnp.concatenate([pp[0] for pp in parts], axis=0)
            st_stack = jnp.concatenate([pp[1] for pp in parts], axis=0)
            if g == 0:
                cp_ew.wait()
                for e in range(E_LOCAL):
                    ewb_ref[pl.ds(e * D_IN, D_IN), :] = (
                        ewf_ref[e].astype(jnp.bfloat16)
                    )
            y_g = jnp.dot(xg_stack, ewb_ref[...],
                          preferred_element_type=jnp.float32)
            sb_ref[g] = jnp.concatenate(
                [y_g.astype(jnp.bfloat16), st_stack], axis=1)
            for j, off in enumerate(offs):
                if off >= N_DEV:
                    continue
                rdma = pltpu.make_async_remote_copy(
                    src_ref=sb_ref.at[g, pl.ds(j * CAP, CAP), :],
                    dst_ref=rb_ref.at[N_DEV - 1 - off],
                    send_sem=ssem.at[off - 1],
                    recv_sem=rsem.at[N_DEV - 1 - off],
                    device_id=((d + off) % N_DEV,),
                    device_id_type=pl.DeviceIdType.MESH,
                )
                rdma.start()
                rdmas.append(rdma)

        cp_sw.wait()
        swb_ref[...] = swf_ref[...].astype(jnp.bfloat16)
        tot = jnp.dot(xb_ref[pl.ds(d * CHUNK, CHUNK), :], swb_ref[...],
                      preferred_element_type=jnp.float32)

        def decomp(blk):
            return _tdot(blk[:, D_OUT:], blk[:, :D_OUT])

        for grp_j in range(3):
            for j in range(4 * grp_j, 4 * grp_j + 4):
                rdmas[j].wait()
            lo = 14 - (4 * grp_j + 3)
            blk = jnp.reshape(rb_ref[pl.ds(lo, 4), :, :], (4 * CAP, PAY))
            tot += decomp(blk)
        for j in range(12, 15):
            rdmas[j].wait()
        last = jnp.concatenate(
            [jnp.reshape(rb_ref[pl.ds(0, 3), :, :], (3 * CAP, PAY)),
             sb_ref[3, pl.ds(3 * CAP, CAP), :]],
            axis=0,
        )
        tot += decomp(last)

        ov_ref[...] = tot
        cp_out = pltpu.make_async_copy(ov_ref, out_hbm, csem.at[0])
        cp_out.start()
        cp_out.wait()

    out = pl.pallas_call(
        body,
        out_shape=jax.ShapeDtypeStruct((CHUNK, D_OUT), jnp.float32),
        in_specs=[
            pl.BlockSpec(memory_space=_HBM),
            pl.BlockSpec(memory_space=pltpu.VMEM),
            pl.BlockSpec(memory_space=pltpu.VMEM),
            pl.BlockSpec(memory_space=_HBM),
            pl.BlockSpec(memory_space=_HBM),
        ],
        out_specs=pl.BlockSpec(memory_space=_HBM),
        scratch_shapes=[
            pltpu.VMEM((N_TOK, D_IN), jnp.float32),
            pltpu.VMEM((N_TOK, D_IN), jnp.bfloat16),
            pltpu.VMEM((E_LOCAL, D_IN, D_OUT), jnp.float32),
            pltpu.VMEM((E_LOCAL * D_IN, D_OUT), jnp.bfloat16),
            pltpu.VMEM((D_IN, D_OUT), jnp.float32),
            pltpu.VMEM((D_IN, D_OUT), jnp.bfloat16),
            pltpu.VMEM((N_TOK, 64), jnp.float32),
            pltpu.VMEM((CHUNK, D_OUT), jnp.float32),
            pltpu.VMEM((GRP, GRP * CAP, PAY), jnp.bfloat16),
            pltpu.VMEM((N_DEV - 1, CAP, PAY), jnp.bfloat16),
            pltpu.SemaphoreType.DMA((N_DEV - 1,)),
            pltpu.SemaphoreType.DMA((N_DEV - 1,)),
            pltpu.SemaphoreType.DMA((3,)),
        ],
        compiler_params=pltpu.CompilerParams(collective_id=0),
    )(x, rw_t, onehot, expert_W, shared_W)
    return out
